# Initial kernel scaffold; baseline (speedup 1.0000x reference)
import functools

import jax
import jax.numpy as jnp
from jax import lax
from jax.experimental import pallas as pl
from jax.experimental.pallas import tpu as pltpu
from jax.experimental.pallas import tpu_sc as plsc

_NUM_AUGS = 1000
_EMBED = 64
_HID = 256
_B = 16384

_NC = 2
_NS = 16
_NW = _NC * _NS
_BPW = _B // _NW

_BS = 512


def _gather_body(aug_hbm, cls_hbm, idx_hbm, o0, o1, o2, o3,
                 idx_v, rows_v, sem):
    wid = lax.axis_index("s") * _NC + lax.axis_index("c")
    base = wid * _BPW
    outs = (o0, o1, o2, o3)
    tables = (aug_hbm, aug_hbm, aug_hbm, cls_hbm)
    for j in range(4):
        pltpu.sync_copy(idx_hbm.at[pl.ds(j * _B + base, _BPW)], idx_v)
        pltpu.async_copy(tables[j].at[idx_v], rows_v, sem).wait()
        pltpu.sync_copy(rows_v, outs[j].at[pl.ds(base, _BPW)])


def _mlp_body(g0, g1, g2, gc, W0, b0, W1, b1, W2, b2, Wout, bout, o_ref):
    f32 = jnp.float32
    hi = jax.lax.Precision.HIGHEST
    aug = g0[...] + g1[...] + g2[...]
    h = jnp.concatenate([aug, gc[...]], axis=1)
    h = lax.dot_general(h, W0[...], (((1,), (1,)), ((), ())),
                        precision=hi, preferred_element_type=f32)
    h = jnp.maximum(h + b0[...], 0.0)
    h = lax.dot_general(h, W1[...], (((1,), (1,)), ((), ())),
                        precision=hi, preferred_element_type=f32)
    h = jnp.maximum(h + b1[...], 0.0)
    h = lax.dot_general(h, W2[...], (((1,), (1,)), ((), ())),
                        precision=hi, preferred_element_type=f32)
    h = jnp.maximum(h + b2[...], 0.0)
    y = lax.dot_general(h, Wout[...], (((1,), (1,)), ((), ())),
                        precision=hi, preferred_element_type=f32)
    o_ref[...] = y + bout[...]


def kernel(x, aug_table, cls_table, W0, b0, W1, b1, W2, b2, Wout, bout):
    aug_z = aug_table.at[_NUM_AUGS - 1].set(0.0)
    idx_flat = x.T.reshape(-1)

    mesh = plsc.VectorSubcoreMesh(core_axis_name="c", subcore_axis_name="s")
    emb = jax.ShapeDtypeStruct((_B, _EMBED), jnp.float32)
    gather = pl.kernel(
        _gather_body,
        mesh=mesh,
        out_type=[emb, emb, emb, emb],
        scratch_types=[
            pltpu.VMEM((_BPW,), jnp.int32),
            pltpu.VMEM((_BPW, _EMBED), jnp.float32),
            pltpu.SemaphoreType.DMA,
        ],
    )
    g0, g1, g2, gc = gather(aug_z, cls_table, idx_flat)

    nblk = _B // _BS
    gspec = pl.BlockSpec((_BS, _EMBED), lambda i: (i, 0))
    wspec = lambda r, c: pl.BlockSpec((r, c), lambda i: (0, 0))
    y = pl.pallas_call(
        _mlp_body,
        grid=(nblk,),
        in_specs=[
            gspec, gspec, gspec, gspec,
            wspec(_HID, 2 * _EMBED),
            wspec(1, _HID),
            wspec(_HID, _HID),
            wspec(1, _HID),
            wspec(_HID, _HID),
            wspec(1, _HID),
            wspec(1, _HID),
            wspec(1, 1),
        ],
        out_specs=pl.BlockSpec((_BS, 1), lambda i: (i, 0)),
        out_shape=jax.ShapeDtypeStruct((_B, 1), jnp.float32),
    )(g0, g1, g2, gc,
      W0, b0.reshape(1, _HID),
      W1, b1.reshape(1, _HID),
      W2, b2.reshape(1, _HID),
      Wout, bout.reshape(1, 1))
    return y


# speedup vs baseline: 1.0810x; 1.0810x over previous
import functools

import jax
import jax.numpy as jnp
from jax import lax
from jax.experimental import pallas as pl
from jax.experimental.pallas import tpu as pltpu
from jax.experimental.pallas import tpu_sc as plsc

_NUM_AUGS = 1000
_NUM_CLS = 100000
_EMBED = 64
_HID = 256
_B = 16384

_NC = 2
_NS = 16
_NW = _NC * _NS
_BPW = _B // _NW

_BS = 512


def _gather_body(aug_hbm, cls_hbm, idx_hbm, o0, o1, o2, o3,
                 idx_v, rows_v, sem):
    wid = lax.axis_index("s") * _NC + lax.axis_index("c")
    base = wid * _BPW
    outs = (o0, o1, o2, o3)
    tables = (aug_hbm, aug_hbm, aug_hbm, cls_hbm)
    for j in range(4):
        pltpu.sync_copy(idx_hbm.at[pl.ds(j * _B + base, _BPW)], idx_v)
        pltpu.async_copy(tables[j].at[idx_v], rows_v, sem).wait()
        pltpu.sync_copy(rows_v, outs[j].at[pl.ds(base, _BPW)])


def _mlp_body(g0, g1, g2, gc, p0, p1, p2, p3,
              W0, b0, W1, b1, W2, b2, Wout, o_ref):
    f32 = jnp.float32
    hi = jax.lax.Precision.HIGHEST

    def half(g, p):
        lo = g[:, :_EMBED]
        hc = g[:, _EMBED:]
        pv = p[...]
        return lo + (hc - lo) * pv

    aug = half(g0, p0) + half(g1, p1) + half(g2, p2)
    h = jnp.concatenate([aug, half(gc, p3)], axis=1)
    h = lax.dot_general(h, W0[...], (((1,), (1,)), ((), ())),
                        precision=hi, preferred_element_type=f32)
    h = jnp.maximum(h + b0[...], 0.0)
    h = lax.dot_general(h, W1[...], (((1,), (1,)), ((), ())),
                        precision=hi, preferred_element_type=f32)
    h = jnp.maximum(h + b1[...], 0.0)
    h = lax.dot_general(h, W2[...], (((1,), (1,)), ((), ())),
                        precision=hi, preferred_element_type=f32)
    h = jnp.maximum(h + b2[...], 0.0)
    o_ref[...] = jnp.sum(h * Wout[...], axis=1, keepdims=True)


def kernel(x, aug_table, cls_table, W0, b0, W1, b1, W2, b2, Wout, bout):
    aug_z = aug_table.at[_NUM_AUGS - 1].set(0.0)
    aug2 = aug_z.reshape(_NUM_AUGS // 2, 2 * _EMBED)
    cls2 = cls_table.reshape(_NUM_CLS // 2, 2 * _EMBED)
    idx_half = (x >> 1).T.reshape(-1)
    par = (x & 1).astype(jnp.float32)

    mesh = plsc.VectorSubcoreMesh(core_axis_name="c", subcore_axis_name="s")
    emb = jax.ShapeDtypeStruct((_B, 2 * _EMBED), jnp.float32)
    gather = pl.kernel(
        _gather_body,
        mesh=mesh,
        out_type=[emb, emb, emb, emb],
        scratch_types=[
            pltpu.VMEM((_BPW,), jnp.int32),
            pltpu.VMEM((_BPW, 2 * _EMBED), jnp.float32),
            pltpu.SemaphoreType.DMA,
        ],
    )
    g0, g1, g2, gc = gather(aug2, cls2, idx_half)

    nblk = _B // _BS
    gspec = pl.BlockSpec((_BS, 2 * _EMBED), lambda i: (i, 0))
    pspec = pl.BlockSpec((_BS, 1), lambda i: (i, 0))
    wspec = lambda r, c: pl.BlockSpec((r, c), lambda i: (0, 0))
    y = pl.pallas_call(
        _mlp_body,
        grid=(nblk,),
        in_specs=[
            gspec, gspec, gspec, gspec,
            pspec, pspec, pspec, pspec,
            wspec(_HID, 2 * _EMBED),
            wspec(1, _HID),
            wspec(_HID, _HID),
            wspec(1, _HID),
            wspec(_HID, _HID),
            wspec(1, _HID),
            wspec(1, _HID),
        ],
        out_specs=pl.BlockSpec((_BS, 1), lambda i: (i, 0)),
        out_shape=jax.ShapeDtypeStruct((_B, 1), jnp.float32),
    )(g0, g1, g2, gc,
      par[:, 0:1], par[:, 1:2], par[:, 2:3], par[:, 3:4],
      W0, b0.reshape(1, _HID),
      W1, b1.reshape(1, _HID),
      W2, b2.reshape(1, _HID),
      Wout)
    return y + bout


# speedup vs baseline: 1.4611x; 1.3516x over previous
import functools

import jax
import jax.numpy as jnp
from jax import lax
from jax.experimental import pallas as pl
from jax.experimental.pallas import tpu as pltpu
from jax.experimental.pallas import tpu_sc as plsc

_NUM_AUGS = 1000
_NUM_CLS = 100000
_EMBED = 64
_HID = 256
_B = 16384

_NC = 2
_NS = 16
_NW = _NC * _NS

_NCHUNK = 4
_CB = _B // _NCHUNK
_BPW = _CB // _NW

_BS = 512


def _gather_body(chunk, aug_hbm, cls_hbm, idx_hbm, o0, o1, o2, o3,
                 idx_v, rows_v, sem):
    wid = lax.axis_index("s") * _NC + lax.axis_index("c")
    base = wid * _BPW
    outs = (o0, o1, o2, o3)
    tables = (aug_hbm, aug_hbm, aug_hbm, cls_hbm)
    for j in range(4):
        pltpu.sync_copy(
            idx_hbm.at[pl.ds(j * _B + chunk * _CB + base, _BPW)], idx_v)
        pltpu.async_copy(tables[j].at[idx_v], rows_v, sem).wait()
        pltpu.sync_copy(rows_v, outs[j].at[pl.ds(base, _BPW)])


def _mlp_body(g0, g1, g2, gc, p0, p1, p2, p3,
              W0, b0, W1, b1, W2, b2, Wout, o_ref):
    f32 = jnp.float32
    hi = jax.lax.Precision.DEFAULT

    def half(g, p):
        lo = g[:, :_EMBED]
        hc = g[:, _EMBED:]
        return lo + (hc - lo) * p[...]

    aug = half(g0, p0) + half(g1, p1) + half(g2, p2)
    h = jnp.concatenate([aug, half(gc, p3)], axis=1)
    h = lax.dot_general(h, W0[...], (((1,), (1,)), ((), ())),
                        precision=hi, preferred_element_type=f32)
    h = jnp.maximum(h + b0[...], 0.0)
    h = lax.dot_general(h, W1[...], (((1,), (1,)), ((), ())),
                        precision=hi, preferred_element_type=f32)
    h = jnp.maximum(h + b1[...], 0.0)
    h = lax.dot_general(h, W2[...], (((1,), (1,)), ((), ())),
                        precision=hi, preferred_element_type=f32)
    h = jnp.maximum(h + b2[...], 0.0)
    o_ref[...] = jnp.sum(h * Wout[...], axis=1, keepdims=True)


def kernel(x, aug_table, cls_table, W0, b0, W1, b1, W2, b2, Wout, bout):
    aug_z = aug_table.at[_NUM_AUGS - 1].set(0.0)
    aug2 = aug_z.reshape(_NUM_AUGS // 2, 2 * _EMBED)
    cls2 = cls_table.reshape(_NUM_CLS // 2, 2 * _EMBED)
    idx_half = (x >> 1).T.reshape(-1)
    par = (x & 1).astype(jnp.float32)

    mesh = plsc.VectorSubcoreMesh(core_axis_name="c", subcore_axis_name="s")
    emb = jax.ShapeDtypeStruct((_CB, 2 * _EMBED), jnp.float32)
    scratch = [
        pltpu.VMEM((_BPW,), jnp.int32),
        pltpu.VMEM((_BPW, 2 * _EMBED), jnp.float32),
        pltpu.SemaphoreType.DMA,
    ]

    nblk = _CB // _BS
    gspec = pl.BlockSpec((_BS, 2 * _EMBED), lambda i: (i, 0))
    pspec = pl.BlockSpec((_BS, 1), lambda i: (i, 0))
    wspec = lambda r, c: pl.BlockSpec((r, c), lambda i: (0, 0))
    mlp = pl.pallas_call(
        _mlp_body,
        grid=(nblk,),
        in_specs=[
            gspec, gspec, gspec, gspec,
            pspec, pspec, pspec, pspec,
            wspec(_HID, 2 * _EMBED),
            wspec(1, _HID),
            wspec(_HID, _HID),
            wspec(1, _HID),
            wspec(_HID, _HID),
            wspec(1, _HID),
            wspec(1, _HID),
        ],
        out_specs=pl.BlockSpec((_BS, 1), lambda i: (i, 0)),
        out_shape=jax.ShapeDtypeStruct((_CB, 1), jnp.float32),
    )

    b0r = b0.reshape(1, _HID)
    b1r = b1.reshape(1, _HID)
    b2r = b2.reshape(1, _HID)

    ys = []
    for c in range(_NCHUNK):
        gather = pl.kernel(
            functools.partial(_gather_body, c),
            mesh=mesh,
            out_type=[emb, emb, emb, emb],
            scratch_types=scratch,
        )
        g0, g1, g2, gc = gather(aug2, cls2, idx_half)
        pc = par[c * _CB:(c + 1) * _CB]
        ys.append(mlp(g0, g1, g2, gc,
                      pc[:, 0:1], pc[:, 1:2], pc[:, 2:3], pc[:, 3:4],
                      W0, b0r, W1, b1r, W2, b2r, Wout))
    return jnp.concatenate(ys, axis=0) + bout


# speedup vs baseline: 1.4745x; 1.0091x over previous
import functools

import jax
import jax.numpy as jnp
from jax import lax
from jax.experimental import pallas as pl
from jax.experimental.pallas import tpu as pltpu
from jax.experimental.pallas import tpu_sc as plsc

_NUM_AUGS = 1000
_NUM_CLS = 100000
_EMBED = 64
_HID = 256
_B = 16384

_NC = 2
_NS = 16
_NW = _NC * _NS

_NCHUNK = 4
_CB = _B // _NCHUNK
_BPW = _CB // _NW

_BS = 1024


def _gather_aug_body(chunk, aug_hbm, idx_hbm, o0, o1, o2,
                     idx_v, rows_v, sem):
    wid = lax.axis_index("s") * _NC + lax.axis_index("c")
    base = wid * _BPW
    outs = (o0, o1, o2)
    for j in range(3):
        pltpu.sync_copy(
            idx_hbm.at[pl.ds(j * _B + chunk * _CB + base, _BPW)], idx_v)
        pltpu.async_copy(aug_hbm.at[idx_v], rows_v, sem).wait()
        pltpu.sync_copy(rows_v, outs[j].at[pl.ds(base, _BPW)])


def _gather_cls_body(chunk, cls_hbm, idx_hbm, oc, idx_v, rows_v, sem):
    wid = lax.axis_index("s") * _NC + lax.axis_index("c")
    base = wid * _BPW
    pltpu.sync_copy(
        idx_hbm.at[pl.ds(3 * _B + chunk * _CB + base, _BPW)], idx_v)
    pltpu.async_copy(cls_hbm.at[idx_v], rows_v, sem).wait()
    pltpu.sync_copy(rows_v, oc.at[pl.ds(base, _BPW)])


def _mlp_body(g0, g1, g2, gc, p0, p1, p2, p3,
              W0, b0, W1, b1, W2, b2, Wout, o_ref):
    f32 = jnp.float32
    hi = jax.lax.Precision.DEFAULT

    def half(g, p):
        lo = g[:, :_EMBED]
        hc = g[:, _EMBED:]
        return lo + (hc - lo) * p[...]

    aug = half(g0, p0) + half(g1, p1) + half(g2, p2)
    h = jnp.concatenate([aug, half(gc, p3)], axis=1)
    h = lax.dot_general(h, W0[...], (((1,), (1,)), ((), ())),
                        precision=hi, preferred_element_type=f32)
    h = jnp.maximum(h + b0[...], 0.0)
    h = lax.dot_general(h, W1[...], (((1,), (1,)), ((), ())),
                        precision=hi, preferred_element_type=f32)
    h = jnp.maximum(h + b1[...], 0.0)
    h = lax.dot_general(h, W2[...], (((1,), (1,)), ((), ())),
                        precision=hi, preferred_element_type=f32)
    h = jnp.maximum(h + b2[...], 0.0)
    o_ref[...] = jnp.sum(h * Wout[...], axis=1, keepdims=True)


def kernel(x, aug_table, cls_table, W0, b0, W1, b1, W2, b2, Wout, bout):
    aug_z = aug_table.at[_NUM_AUGS - 1].set(0.0)
    aug2 = aug_z.reshape(_NUM_AUGS // 2, 2 * _EMBED)
    cls2 = cls_table.reshape(_NUM_CLS // 2, 2 * _EMBED)
    idx_half = (x >> 1).T.reshape(-1)
    par = (x & 1).astype(jnp.float32)

    mesh = plsc.VectorSubcoreMesh(core_axis_name="c", subcore_axis_name="s")
    emb = jax.ShapeDtypeStruct((_CB, 2 * _EMBED), jnp.float32)
    scratch = [
        pltpu.VMEM((_BPW,), jnp.int32),
        pltpu.VMEM((_BPW, 2 * _EMBED), jnp.float32),
        pltpu.SemaphoreType.DMA,
    ]

    nblk = _CB // _BS
    gspec = pl.BlockSpec((_BS, 2 * _EMBED), lambda i: (i, 0))
    pspec = pl.BlockSpec((_BS, 1), lambda i: (i, 0))
    wspec = lambda r, c: pl.BlockSpec((r, c), lambda i: (0, 0))
    mlp = pl.pallas_call(
        _mlp_body,
        grid=(nblk,),
        in_specs=[
            gspec, gspec, gspec, gspec,
            pspec, pspec, pspec, pspec,
            wspec(_HID, 2 * _EMBED),
            wspec(1, _HID),
            wspec(_HID, _HID),
            wspec(1, _HID),
            wspec(_HID, _HID),
            wspec(1, _HID),
            wspec(1, _HID),
        ],
        out_specs=pl.BlockSpec((_BS, 1), lambda i: (i, 0)),
        out_shape=jax.ShapeDtypeStruct((_CB, 1), jnp.float32),
    )

    b0r = b0.reshape(1, _HID)
    b1r = b1.reshape(1, _HID)
    b2r = b2.reshape(1, _HID)

    aug_gathered = []
    for c in range(_NCHUNK):
        gather_aug = pl.kernel(
            functools.partial(_gather_aug_body, c),
            mesh=mesh,
            out_type=[emb, emb, emb],
            scratch_types=scratch,
        )
        aug_gathered.append(gather_aug(aug2, idx_half))

    ys = []
    for c in range(_NCHUNK):
        gather_cls = pl.kernel(
            functools.partial(_gather_cls_body, c),
            mesh=mesh,
            out_type=emb,
            scratch_types=scratch,
        )
        gc = gather_cls(cls2, idx_half)
        g0, g1, g2 = aug_gathered[c]
        pc = par[c * _CB:(c + 1) * _CB]
        ys.append(mlp(g0, g1, g2, gc,
                      pc[:, 0:1], pc[:, 1:2], pc[:, 2:3], pc[:, 3:4],
                      W0, b0r, W1, b1r, W2, b2r, Wout))
    return jnp.concatenate(ys, axis=0) + bout
